# Initial kernel scaffold; baseline (speedup 1.0000x reference)
import jax
import jax.numpy as jnp
from jax import lax
from jax.experimental import pallas as pl
from jax.experimental.pallas import tpu as pltpu
from jax.experimental.pallas import tpu_sc as plsc

N_TOKENS = 204800
N_BAGS = 4096
EMB_DIM = 64
VOCAB = 100000

NUM_CORES = 2
NUM_SUBCORES = 16
NUM_WORKERS = NUM_CORES * NUM_SUBCORES
BAGS_PER_W = N_BAGS // NUM_WORKERS
LANES = 16
DGRP = EMB_DIM // LANES
CHUNK = 128


def _sc_body(input_hbm, offsets_hbm, weights_hbm, table_hbm, out_hbm,
             off_v, idx_v, w_v, rows_v, out_local, sem):
    wid = lax.axis_index("s") * NUM_CORES + lax.axis_index("c")
    b0 = wid * BAGS_PER_W

    pltpu.sync_copy(offsets_hbm, off_v)

    def zero_body(i, carry):
        for j in range(DGRP):
            out_local[i, pl.ds(j * LANES, LANES)] = jnp.zeros(
                (LANES,), jnp.float32)
        return carry

    lax.fori_loop(0, BAGS_PER_W, zero_body, 0)

    def off_at(b):
        v = off_v[jnp.minimum(b, N_BAGS - 1)]
        return jnp.where(b >= N_BAGS, N_TOKENS, v)

    start = off_at(b0)
    end = off_at(b0 + BAGS_PER_W)

    c0 = start // CHUNK
    c1 = (end + CHUNK - 1) // CHUNK

    def chunk_body(c, b):
        tok0 = c * CHUNK
        pltpu.sync_copy(input_hbm.at[pl.ds(tok0, CHUNK)], idx_v)
        pltpu.async_copy(table_hbm.at[idx_v], rows_v, sem).wait()
        pltpu.sync_copy(weights_hbm.at[pl.ds(tok0, CHUNK)], w_v)

        t_lo = jnp.maximum(tok0, start)
        t_hi = jnp.minimum(tok0 + CHUNK, end)

        def tok_body(t, bb):
            bb = lax.while_loop(lambda x: off_at(x + 1) <= t,
                                lambda x: x + 1, bb)
            r = t - tok0
            wgt = w_v[r]
            bl = bb - b0
            for j in range(DGRP):
                sl = pl.ds(j * LANES, LANES)
                out_local[bl, sl] += wgt * rows_v[r, sl]
            return bb

        return lax.fori_loop(t_lo, t_hi, tok_body, b)

    lax.fori_loop(c0, c1, chunk_body, b0)

    pltpu.sync_copy(out_local, out_hbm.at[pl.ds(b0, BAGS_PER_W)])


def kernel(input_, offsets, emb_weights, emb_table):
    mesh = plsc.VectorSubcoreMesh(core_axis_name="c", subcore_axis_name="s")
    f = pl.kernel(
        _sc_body,
        mesh=mesh,
        out_type=jax.ShapeDtypeStruct((N_BAGS, EMB_DIM), jnp.float32),
        scratch_types=[
            pltpu.VMEM((N_BAGS,), jnp.int32),
            pltpu.VMEM((CHUNK,), jnp.int32),
            pltpu.VMEM((CHUNK,), jnp.float32),
            pltpu.VMEM((CHUNK, EMB_DIM), jnp.float32),
            pltpu.VMEM((BAGS_PER_W, EMB_DIM), jnp.float32),
            pltpu.SemaphoreType.DMA,
        ],
    )
    return f(input_.astype(jnp.int32), offsets.astype(jnp.int32),
             emb_weights, emb_table)


# speedup vs baseline: 78.2155x; 78.2155x over previous
import functools

import jax
import jax.numpy as jnp
from jax import lax
from jax.experimental import pallas as pl
from jax.experimental.pallas import tpu as pltpu
from jax.experimental.pallas import tpu_sc as plsc

N_TOKENS = 204800
N_BAGS = 4096
EMB_DIM = 64
VOCAB = 100000

NUM_CORES = 2
NUM_SUBCORES = 16
NUM_WORKERS = NUM_CORES * NUM_SUBCORES
LANES = 16
DGRP = EMB_DIM // LANES

TOK_PER_W = N_TOKENS // NUM_WORKERS
TOK_PER_T = N_TOKENS // NUM_SUBCORES
BAGS_PER_T = N_BAGS // NUM_SUBCORES
CHUNK = 128
N_CHUNKS = TOK_PER_W // CHUNK


def _zeros16(dtype):
    return jnp.zeros((LANES,), dtype)


def _sc_body(input_hbm, offsets_hbm, weights_hbm, table_hbm, partials_hbm,
             m_shared, s_shared, acc_shared,
             mslice_v, idx2_v, ones_v, rows_v, w_v, cseg_v, s16_v, tvec_v,
             tokidx_v, sem):
    cid = lax.axis_index("c")
    sid = lax.axis_index("s")
    wid = sid * NUM_CORES + cid

    def zero_mslice(k, carry):
        mslice_v[pl.ds(k * LANES, LANES)] = _zeros16(jnp.int32)
        return carry
    lax.fori_loop(0, TOK_PER_T // LANES, zero_mslice, 0)
    pltpu.sync_copy(mslice_v, m_shared.at[pl.ds(sid * TOK_PER_T, TOK_PER_T)])

    def zero_rows(i, carry):
        for j in range(DGRP):
            rows_v[i, pl.ds(j * LANES, LANES)] = _zeros16(jnp.float32)
        return carry
    lax.fori_loop(0, CHUNK, zero_rows, 0)
    for half in range(BAGS_PER_T // CHUNK):
        pltpu.sync_copy(
            rows_v,
            acc_shared.at[pl.ds(sid * BAGS_PER_T + half * CHUNK, CHUNK)])

    for j in range(BAGS_PER_T // CHUNK):
        pltpu.sync_copy(
            offsets_hbm.at[pl.ds(sid * BAGS_PER_T + j * CHUNK, CHUNK)],
            idx2_v.at[j])
    def fill_ones(k, carry):
        ones_v[pl.ds(k * LANES, LANES)] = jnp.full((LANES,), 1, jnp.int32)
        return carry
    lax.fori_loop(0, CHUNK // LANES, fill_ones, 0)

    plsc.subcore_barrier()

    for j in range(BAGS_PER_T // CHUNK):
        pltpu.sync_copy(ones_v, m_shared.at[idx2_v.at[j]], add=True)

    plsc.subcore_barrier()

    pltpu.sync_copy(m_shared.at[pl.ds(sid * TOK_PER_T, TOK_PER_T)], mslice_v)

    def cumsum_body(k, carry):
        v = mslice_v[pl.ds(k * LANES, LANES)]
        cs = plsc.cumsum(v) + carry
        mslice_v[pl.ds(k * LANES, LANES)] = cs
        return cs[LANES - 1]
    total = lax.fori_loop(0, TOK_PER_T // LANES, cumsum_body,
                          jnp.int32(0))

    pltpu.sync_copy(mslice_v, m_shared.at[pl.ds(sid * TOK_PER_T, TOK_PER_T)])
    tvec_v[pl.ds(0, LANES)] = jnp.full((LANES,), 1, jnp.int32) * total
    pltpu.sync_copy(tvec_v, s_shared.at[sid])

    plsc.subcore_barrier()

    pltpu.sync_copy(s_shared, s16_v)
    owner = wid * TOK_PER_W // TOK_PER_T
    base = jnp.int32(0)
    for i in range(NUM_SUBCORES):
        ti = s16_v[i, pl.ds(0, LANES)][0]
        base = base + jnp.where(i < owner, ti, jnp.int32(0))
    seg_bias = base - 1

    def chunk_body(k, carry):
        tok0 = wid * TOK_PER_W + k * CHUNK
        pltpu.sync_copy(input_hbm.at[pl.ds(tok0, CHUNK)], tokidx_v)
        pltpu.async_copy(table_hbm.at[tokidx_v], rows_v, sem).wait()
        pltpu.sync_copy(weights_hbm.at[pl.ds(tok0, CHUNK)],
                        w_v.at[pl.ds(0, CHUNK)])
        pltpu.sync_copy(m_shared.at[pl.ds(tok0, CHUNK)], cseg_v.at[0])

        for g in range(CHUNK // LANES):
            sl = pl.ds(g * LANES, LANES)
            cseg_v[0, sl] = cseg_v[0, sl] + seg_bias

        def tok_body(r, carry2):
            wgt = w_v[pl.ds(r, LANES)][0]
            for j in range(DGRP):
                sl = pl.ds(j * LANES, LANES)
                rows_v[r, sl] = rows_v[r, sl] * wgt
            return carry2
        lax.fori_loop(0, CHUNK, tok_body, 0)

        pltpu.sync_copy(rows_v, acc_shared.at[cseg_v.at[0]], add=True)
        return carry

    lax.fori_loop(0, N_CHUNKS, chunk_body, 0)

    plsc.subcore_barrier()

    for half in range(BAGS_PER_T // CHUNK):
        row0 = sid * BAGS_PER_T + half * CHUNK
        pltpu.sync_copy(acc_shared.at[pl.ds(row0, CHUNK)],
                        partials_hbm.at[cid, pl.ds(row0, CHUNK)])


def _combine_body(p_ref, o_ref):
    o_ref[...] = p_ref[0] + p_ref[1]


def kernel(input_, offsets, emb_weights, emb_table):
    mesh = plsc.VectorSubcoreMesh(core_axis_name="c", subcore_axis_name="s")
    sc_fn = pl.kernel(
        _sc_body,
        mesh=mesh,
        out_type=jax.ShapeDtypeStruct((NUM_CORES, N_BAGS, EMB_DIM),
                                      jnp.float32),
        scratch_types=[
            pltpu.VMEM_SHARED((N_TOKENS,), jnp.int32),
            pltpu.VMEM_SHARED((NUM_SUBCORES, LANES), jnp.int32),
            pltpu.VMEM_SHARED((N_BAGS, EMB_DIM), jnp.float32),
            pltpu.VMEM((TOK_PER_T,), jnp.int32),
            pltpu.VMEM((BAGS_PER_T // CHUNK, CHUNK), jnp.int32),
            pltpu.VMEM((CHUNK,), jnp.int32),
            pltpu.VMEM((CHUNK, EMB_DIM), jnp.float32),
            pltpu.VMEM((CHUNK + LANES,), jnp.float32),
            pltpu.VMEM((1, CHUNK), jnp.int32),
            pltpu.VMEM((NUM_SUBCORES, LANES), jnp.int32),
            pltpu.VMEM((LANES,), jnp.int32),
            pltpu.VMEM((CHUNK,), jnp.int32),
            pltpu.SemaphoreType.DMA,
        ],
        compiler_params=pltpu.CompilerParams(needs_layout_passes=False,
                                             use_tc_tiling_on_sc=False),
    )
    partials = sc_fn(input_.astype(jnp.int32), offsets.astype(jnp.int32),
                     emb_weights, emb_table)

    combine = pl.pallas_call(
        _combine_body,
        out_shape=jax.ShapeDtypeStruct((N_BAGS, EMB_DIM), jnp.float32),
    )
    return combine(partials)
